# baseline (speedup 1.0000x reference)
import functools

import jax
import jax.numpy as jnp
from jax import lax
from jax.experimental import pallas as pl
from jax.experimental.pallas import tpu as pltpu
from jax.experimental.pallas import tpu_sc as plsc

_BATCH = 16384
_H_DIM = 32
_NW = 32
_NCOLS = 7813
_Q = 245
_SPAN = 252
_CH = 4
_NPAIR = 32
_MAXOFF = _NCOLS - _CH
_C = 640
_NGRP = _C // 16
_FL = 64
_PIECE = 4096
_NPIECE = _BATCH // _PIECE
_OUT_ROWS = _BATCH + _NW


def _body(
    idx_hbm,
    table_hbm,
    out_hbm,
    piece_v,
    rbuf,
    bbuf,
    rows_v,
    rows_w,
    stage_a,
    stage_b,
    sem_a,
    sem_b,
    sem_o,
):
    wid = lax.axis_index("s") * 2 + lax.axis_index("c")
    lo = jnp.minimum(wid * _Q, _NCOLS)
    lanes16 = jax.lax.iota(jnp.int32, 16)
    sink = jnp.full((16,), _BATCH, jnp.int32) + wid

    def filter_pass(t):
        for g in range(_C // 16):
            bbuf[pl.ds(g * 16, 16)] = sink

        def piece_step(p, cnt):
            pltpu.sync_copy(idx_hbm.at[pl.ds(p * _PIECE, _PIECE)], piece_v)

            def grp(g, cnt):
                rv = piece_v[pl.ds(pl.multiple_of(g * 16, 16), 16)]
                tc = rv >> 7
                m = (tc >= lo) & (tc < lo + _SPAN)
                mi = m.astype(jnp.int32)
                ps = plsc.cumsum(mi)
                pos = cnt + ps - 1 - t * _C
                mw = m & (pos >= 0) & (pos < _C)
                posc = jnp.clip(pos, 0, _C - 1)
                bv = lanes16 + (p * _PIECE + g * 16)
                plsc.store_scatter(rbuf, [posc], rv, mask=mw)
                plsc.store_scatter(bbuf, [posc], bv, mask=mw)
                return cnt + ps[15]

            return lax.fori_loop(0, _PIECE // 16, grp, cnt, unroll=False)

        return lax.fori_loop(
            0, _NPIECE, piece_step, jnp.zeros((16,), jnp.int32), unroll=False
        )

    def extract_chunk(stage, off_cols, ngrp):
        l0 = off_cols * 128

        def grp(gi, carry):
            rv = rbuf[pl.ds(pl.multiple_of(gi * 16, 16), 16)]
            m2 = (rv >= l0) & (rv < l0 + _CH * 128)

            @pl.when(jnp.any(m2))
            def _():
                l = jnp.clip(rv - l0, 0, _CH * 128 - 1)
                kv = lanes16 + gi * 16
                for c in range(_H_DIM):
                    cc = jnp.full((16,), c, jnp.int32)
                    vals = plsc.load_gather(stage, [cc, l], mask=m2)
                    plsc.store_scatter(rows_v, [kv, cc], vals, mask=m2)

            return carry

        lax.fori_loop(0, ngrp, grp, 0, unroll=False)


    def round_cond(c):
        t, cnt_s = c
        return t * _C < cnt_s

    def round_step(c):
        t, _ = c
        cnt_s = filter_pass(t)[0]
        ngrp = (jnp.minimum(cnt_s - t * _C, _C) + 15) // 16

        def pair_step(i, carry):
            off_a = jnp.minimum(lo + (2 * i) * _CH, _MAXOFF)
            off_b = jnp.minimum(lo + (2 * i + 1) * _CH, _MAXOFF)
            cp_a = pltpu.async_copy(
                table_hbm.at[
                    :, pl.ds(pl.multiple_of(off_a * 128, 128), _CH * 128)
                ],
                stage_a,
                sem_a,
            )
            cp_b = pltpu.async_copy(
                table_hbm.at[
                    :, pl.ds(pl.multiple_of(off_b * 128, 128), _CH * 128)
                ],
                stage_b,
                sem_b,
            )
            cp_a.wait()
            extract_chunk(stage_a, off_a, ngrp)
            cp_b.wait()
            extract_chunk(stage_b, off_b, ngrp)
            return carry

        lax.fori_loop(0, _NPAIR, pair_step, 0, unroll=False)

        for sub in range(_C // _FL):
            @pl.when(sub * _FL < cnt_s - t * _C)
            def _():
                def widen(g, carry):
                    jv = lanes16 + (sub * _FL + g * 16)
                    jl = lanes16 + g * 16
                    for c in range(_H_DIM):
                        cc = jnp.full((16,), c, jnp.int32)
                        vals = plsc.load_gather(rows_v, [jv, cc])
                        plsc.store_scatter(rows_w, [jl, cc], vals)
                    return carry

                lax.fori_loop(0, _FL // 16, widen, 0, unroll=False)
                pltpu.async_copy(
                    rows_w,
                    out_hbm.at[bbuf.at[pl.ds(sub * _FL, _FL)]],
                    sem_o,
                ).wait()
        return (t + 1, cnt_s)

    lax.while_loop(round_cond, round_step, (jnp.int32(0), jnp.int32(1)))


@jax.jit
def _lookup(idx, table_t):
    mesh = plsc.VectorSubcoreMesh(core_axis_name="c", subcore_axis_name="s")
    out = pl.kernel(
        _body,
        out_type=jax.ShapeDtypeStruct((_OUT_ROWS, 128), jnp.float32),
        mesh=mesh,
        scratch_types=[
            pltpu.VMEM((_PIECE,), jnp.int32),
            pltpu.VMEM((_C,), jnp.int32),
            pltpu.VMEM((_C,), jnp.int32),
            pltpu.VMEM((_C, _H_DIM), jnp.float32),
            pltpu.VMEM((_FL, 128), jnp.float32),
            pltpu.VMEM((_H_DIM, _CH * 128), jnp.float32),
            pltpu.VMEM((_H_DIM, _CH * 128), jnp.float32),
            pltpu.SemaphoreType.DMA,
            pltpu.SemaphoreType.DMA,
            pltpu.SemaphoreType.DMA,
        ],
        compiler_params=pltpu.CompilerParams(needs_layout_passes=False),
    )(idx, table_t)
    return out[:_BATCH, :_H_DIM]


def kernel(g, h, r, norm, emb_weight):
    idx = h.astype(jnp.int32).reshape(_BATCH)
    return _lookup(idx, emb_weight.T)


# speedup vs baseline: 1.2625x; 1.2625x over previous
import functools

import jax
import jax.numpy as jnp
from jax import lax
from jax.experimental import pallas as pl
from jax.experimental.pallas import tpu as pltpu
from jax.experimental.pallas import tpu_sc as plsc

_BATCH = 16384
_H_DIM = 32
_NW = 32
_NCOLS = 7813
_Q = 245
_SPAN = 252
_CH = 4
_NPAIR = 32
_MAXOFF = _NCOLS - _CH
_C = 640
_NGRP = _C // 16
_FL = 64
_PIECE = 4096
_NPIECE = _BATCH // _PIECE
_OUT_ROWS = _BATCH + _NW


def _body(
    idx_hbm,
    table_hbm,
    out_hbm,
    piece_v,
    rbuf,
    bbuf,
    rows_v,
    rows_w,
    stage_a,
    stage_b,
    sem_a,
    sem_b,
    sem_o,
):
    wid = lax.axis_index("s") * 2 + lax.axis_index("c")
    lo = jnp.minimum(wid * _Q, _NCOLS)
    lanes16 = jax.lax.iota(jnp.int32, 16)
    sink = jnp.full((16,), _BATCH, jnp.int32) + wid

    def filter_pass(t):
        for g in range(_C // 16):
            bbuf[pl.ds(g * 16, 16)] = sink

        def piece_step(p, cnt):
            pltpu.sync_copy(idx_hbm.at[pl.ds(p * _PIECE, _PIECE)], piece_v)

            def grp(g, cnt):
                rv = piece_v[pl.ds(pl.multiple_of(g * 16, 16), 16)]
                tc = rv >> 7
                m = (tc >= lo) & (tc < lo + _SPAN)
                mi = m.astype(jnp.int32)
                ps = plsc.cumsum(mi)
                pos = cnt + ps - 1 - t * _C
                mw = m & (pos >= 0) & (pos < _C)
                posc = jnp.clip(pos, 0, _C - 1)
                bv = lanes16 + (p * _PIECE + g * 16)
                plsc.store_scatter(rbuf, [posc], rv, mask=mw)
                plsc.store_scatter(bbuf, [posc], bv, mask=mw)
                return cnt + ps[15]

            return lax.fori_loop(0, _PIECE // 16, grp, cnt, unroll=2)

        return lax.fori_loop(
            0, _NPIECE, piece_step, jnp.zeros((16,), jnp.int32), unroll=False
        )

    def extract_chunk(stage, off_cols, ngrp):
        l0 = off_cols * 128

        def grp(gi, carry):
            rv = rbuf[pl.ds(pl.multiple_of(gi * 16, 16), 16)]
            m2 = (rv >= l0) & (rv < l0 + _CH * 128)

            @pl.when(jnp.any(m2))
            def _():
                l = jnp.clip(rv - l0, 0, _CH * 128 - 1)
                kv = lanes16 + gi * 16
                for c in range(_H_DIM):
                    cc = jnp.full((16,), c, jnp.int32)
                    vals = plsc.load_gather(stage, [cc, l], mask=m2)
                    plsc.store_scatter(rows_v, [kv, cc], vals, mask=m2)

            return carry

        lax.fori_loop(0, ngrp, grp, 0, unroll=False)


    def round_cond(c):
        t, cnt_s = c
        return t * _C < cnt_s

    def round_step(c):
        t, _ = c
        cnt_s = filter_pass(t)[0]
        ngrp = (jnp.minimum(cnt_s - t * _C, _C) + 15) // 16

        def off_of(j):
            return jnp.minimum(lo + j * _CH, _MAXOFF)

        def fire(j, stage, sem):
            return pltpu.async_copy(
                table_hbm.at[
                    :, pl.ds(pl.multiple_of(off_of(j) * 128, 128), _CH * 128)
                ],
                stage,
                sem,
            )

        def drain(stage, sem):
            pltpu.make_async_copy(
                table_hbm.at[:, pl.ds(0, _CH * 128)], stage, sem
            ).wait()

        fire(0, stage_a, sem_a)
        fire(1, stage_b, sem_b)

        def pair_step(i, carry):
            drain(stage_a, sem_a)
            extract_chunk(stage_a, off_of(2 * i), ngrp)

            @pl.when(i < _NPAIR - 1)
            def _():
                fire(2 * i + 2, stage_a, sem_a)

            drain(stage_b, sem_b)
            extract_chunk(stage_b, off_of(2 * i + 1), ngrp)

            @pl.when(i < _NPAIR - 1)
            def _():
                fire(2 * i + 3, stage_b, sem_b)

            return carry

        lax.fori_loop(0, _NPAIR, pair_step, 0, unroll=False)

        for sub in range(_C // _FL):
            @pl.when(sub * _FL < cnt_s - t * _C)
            def _():
                def widen(g, carry):
                    jv = lanes16 + (sub * _FL + g * 16)
                    jl = lanes16 + g * 16
                    for c in range(_H_DIM):
                        cc = jnp.full((16,), c, jnp.int32)
                        vals = plsc.load_gather(rows_v, [jv, cc])
                        plsc.store_scatter(rows_w, [jl, cc], vals)
                    return carry

                lax.fori_loop(0, _FL // 16, widen, 0, unroll=False)
                pltpu.async_copy(
                    rows_w,
                    out_hbm.at[bbuf.at[pl.ds(sub * _FL, _FL)]],
                    sem_o,
                ).wait()
        return (t + 1, cnt_s)

    lax.while_loop(round_cond, round_step, (jnp.int32(0), jnp.int32(1)))


@jax.jit
def _lookup(idx, table_t):
    mesh = plsc.VectorSubcoreMesh(core_axis_name="c", subcore_axis_name="s")
    out = pl.kernel(
        _body,
        out_type=jax.ShapeDtypeStruct((_OUT_ROWS, 128), jnp.float32),
        mesh=mesh,
        scratch_types=[
            pltpu.VMEM((_PIECE,), jnp.int32),
            pltpu.VMEM((_C,), jnp.int32),
            pltpu.VMEM((_C,), jnp.int32),
            pltpu.VMEM((_C, _H_DIM), jnp.float32),
            pltpu.VMEM((_FL, 128), jnp.float32),
            pltpu.VMEM((_H_DIM, _CH * 128), jnp.float32),
            pltpu.VMEM((_H_DIM, _CH * 128), jnp.float32),
            pltpu.SemaphoreType.DMA,
            pltpu.SemaphoreType.DMA,
            pltpu.SemaphoreType.DMA,
        ],
        compiler_params=pltpu.CompilerParams(needs_layout_passes=False),
    )(idx, table_t)
    return out[:_BATCH, :_H_DIM]


def kernel(g, h, r, norm, emb_weight):
    idx = h.astype(jnp.int32).reshape(_BATCH)
    return _lookup(idx, emb_weight.T)
